# baseline (speedup 1.0000x reference)
import functools

import numpy as np

import jax
import jax.numpy as jnp
from jax import lax
from jax.experimental import pallas as pl
from jax.experimental.pallas import tpu as pltpu
from jax.experimental.pallas import tpu_sc as plsc

NC = 2
NS = 16
NW = NC * NS
EPB = 128
G = 16


def _sc_mesh():
    return plsc.VectorSubcoreMesh(
        core_axis_name="c", subcore_axis_name="s", num_cores=NC, num_subcores=NS
    )


def _make_degree(NP, S):
    @functools.partial(
        pl.kernel,
        out_type=jax.ShapeDtypeStruct((NW, NP), jnp.float32),
        mesh=_sc_mesh(),
        scratch_types=[
            pltpu.VMEM((S, EPB), jnp.int32),
            pltpu.VMEM((NP,), jnp.float32),
        ],
        compiler_params=pltpu.CompilerParams(needs_layout_passes=False),
    )
    def deg_kernel(dsts_hbm, out_hbm, idx_d, hist):
        c = lax.axis_index("c")
        s = lax.axis_index("s")
        wid = c * NS + s
        pltpu.sync_copy(dsts_hbm.at[wid], idx_d)

        def zbody(i, carry):
            hist[pl.ds(i * 16, 16)] = jnp.zeros((16,), jnp.float32)
            return carry

        lax.fori_loop(0, NP // 16, zbody, 0)
        ones16 = jnp.ones((16,), jnp.float32)

        def body(j, carry):
            for k in range(EPB // 16):
                idx = idx_d[j, pl.ds(k * 16, 16)]
                plsc.addupdate_scatter(hist, [idx], ones16)
            return carry

        lax.fori_loop(0, S, body, 0)
        pltpu.sync_copy(hist, out_hbm.at[wid])

    return deg_kernel


def _make_agg(NP, S):
    rq = NP // 4 // NS

    @functools.partial(
        pl.kernel,
        out_type=jax.ShapeDtypeStruct((NC, NP // 4, 128), jnp.bfloat16),
        mesh=_sc_mesh(),
        scratch_types=[
            pltpu.VMEM((S, EPB), jnp.int32),
            pltpu.VMEM((S, EPB), jnp.int32),
            pltpu.VMEM((EPB, 128), jnp.float32),
            pltpu.VMEM((EPB, 128), jnp.float32),
            pltpu.VMEM((NP // 4 // NS, 128), jnp.float32),
            pltpu.VMEM((NP // 4 // NS, 128), jnp.bfloat16),
            pltpu.VMEM_SHARED((NP // 4, 128), jnp.float32),
            pltpu.SemaphoreType.DMA,
            pltpu.SemaphoreType.DMA,
            pltpu.SemaphoreType.DMA,
            pltpu.SemaphoreType.DMA,
        ],
        compiler_params=pltpu.CompilerParams(needs_layout_passes=False),
    )
    def agg_kernel(tab_hbm, gidx_hbm, didx_hbm, out_hbm,
                   idx_g, idx_d, rows_a, rows_b, stage, stage_bf, acc,
                   sem_a, sem_b, sem_sa, sem_sb):
        c = lax.axis_index("c")
        s = lax.axis_index("s")
        wid = c * NS + s
        row0 = s * rq

        def zbody(i, carry):
            for k in range(8):
                stage[i, pl.ds(k * 16, 16)] = jnp.zeros((16,), jnp.float32)
            return carry

        lax.fori_loop(0, rq, zbody, 0)
        pltpu.sync_copy(stage, acc.at[pl.ds(row0, rq)])
        pltpu.sync_copy(gidx_hbm.at[wid], idx_g)
        pltpu.sync_copy(didx_hbm.at[wid], idx_d)
        plsc.subcore_barrier()

        pltpu.async_copy(tab_hbm.at[idx_g.at[0]], rows_a, sem_a)

        def body(t, carry):
            j0 = 2 * t
            j1 = 2 * t + 1
            pltpu.make_async_copy(tab_hbm.at[idx_g.at[j0]], rows_a, sem_a).wait()
            pltpu.async_copy(rows_a, acc.at[idx_d.at[j0]], sem_sa, add=True)
            pltpu.make_async_copy(tab_hbm.at[idx_g.at[j1]], rows_b, sem_b).wait()
            pltpu.async_copy(rows_b, acc.at[idx_d.at[j1]], sem_sb, add=True)
            pltpu.make_async_copy(rows_a, acc.at[idx_d.at[j0]], sem_sa).wait()

            @pl.when(t + 1 < S // 2)
            def _():
                pltpu.async_copy(tab_hbm.at[idx_g.at[j1 + 1]], rows_a, sem_a)

            pltpu.make_async_copy(rows_b, acc.at[idx_d.at[j1]], sem_sb).wait()

            @pl.when(t + 1 < S // 2)
            def _():
                pltpu.async_copy(tab_hbm.at[idx_g.at[j1 + 2]], rows_b, sem_b)

            return carry

        pltpu.async_copy(tab_hbm.at[idx_g.at[1]], rows_b, sem_b)
        lax.fori_loop(0, S // 2, body, 0)
        plsc.subcore_barrier()
        pltpu.sync_copy(acc.at[pl.ds(row0, rq)], stage)

        def wbody(r, carry):
            for g in range(4):
                v0 = stage[r, pl.ds(g * 32, 16)]
                v1 = stage[r, pl.ds(g * 32 + 16, 16)]
                pk = plsc.pack(v0, v1, format=plsc.PackFormat.INTERLEAVED)
                stage_bf[r, pl.ds(g * 32, 32)] = pk
            return carry

        lax.fori_loop(0, rq, wbody, 0)
        pltpu.sync_copy(stage_bf, out_hbm.at[c, pl.ds(row0, rq)])

    return agg_kernel


def _tc1(deg2_ref, x_ref, w1_ref, dinv_ref, tab_ref):
    deg = jnp.sum(deg2_ref[...], axis=0) + 1.0
    dinv = lax.rsqrt(deg)
    dinv_ref[...] = dinv
    h = jnp.dot(x_ref[...], w1_ref[...], preferred_element_type=jnp.float32)
    hs = h * dinv[:, None]
    f = h.shape[1]
    for q in range(4):
        tab_ref[q] = jnp.pad(hs, ((0, 0), (q * f, 128 - (q + 1) * f)))


def _tcstep(p_ref, hs_ref, dinv_ref, b_ref, w_ref, y_ref, tabn_ref):
    dinv = dinv_ref[...]
    y = dinv[:, None] * (p_ref[...] + hs_ref[...]) + b_ref[...]
    y_ref[...] = y
    h = jnp.maximum(y, 0.0)
    hsn = jnp.dot(h, w_ref[...], preferred_element_type=jnp.float32) * dinv[:, None]
    f = hsn.shape[1]
    for q in range(4):
        tabn_ref[q] = jnp.pad(hsn, ((0, 0), (q * f, 128 - (q + 1) * f)))


def _tc3(y_ref, wn_ref, bn_ref, batch_ref, wg_ref, bg_ref, out_ref):
    node = jnp.dot(y_ref[...], wn_ref[...], preferred_element_type=jnp.float32) + bn_ref[0]
    npad = node.shape[0]
    seg = lax.broadcasted_iota(jnp.int32, (npad, G), 1)
    onehot = (batch_ref[...][:, None] == seg)
    ssum = jnp.sum(jnp.where(onehot, node, 0.0), axis=0)
    cnt = jnp.sum(jnp.where(onehot, 1.0, 0.0), axis=0)
    smax = jnp.max(jnp.where(onehot, node, -jnp.inf), axis=0)
    mean = ssum / jnp.maximum(cnt, 1.0)
    res = ssum * wg_ref[0, 0] + mean * wg_ref[1, 0] + smax * wg_ref[2, 0] + bg_ref[0]
    out_ref[...] = res[:, None]


_IPERM = np.concatenate(
    [32 * g + np.concatenate([2 * np.arange(16), 2 * np.arange(16) + 1])
     for g in range(4)]
)


def kernel(x, edge_index, batch, W1, b1, W2, b2, Wn, bn, Wg, bg):
    N, F_IN = x.shape
    E = edge_index.shape[1]
    F1 = W1.shape[1]
    F2 = W2.shape[1]

    NP = ((N // (NS * EPB)) + 1) * (NS * EPB)
    S = -(-E // (NW * EPB))
    S = ((S + 7) // 8) * 8
    EP = NW * S * EPB

    pad = EP - E
    pad_idx = (N + (jnp.arange(pad, dtype=jnp.int32) % (NP - N))).astype(jnp.int32)
    src_p = jnp.concatenate([edge_index[0], pad_idx])
    dst_p = jnp.concatenate([edge_index[1], pad_idx])
    gidx = ((dst_p % 4) * NP + src_p).reshape(NW, S, EPB)
    didx = (dst_p // 4).reshape(NW, S, EPB)
    dst_p = dst_p.reshape(NW, S, EPB)

    x_p = jnp.pad(x, ((0, NP - N), (0, 0)))
    batch_p = jnp.pad(batch, ((0, NP - N),), constant_values=G)
    W1p = jnp.pad(W1, ((0, 0), (0, F2 - F1)))
    b1p = jnp.pad(b1, ((0, F2 - F1)))
    W2p = jnp.pad(W2, ((0, F2 - F1), (0, 0)))

    deg2 = _make_degree(NP, S)(dst_p)

    dinv, tab1 = pl.pallas_call(
        _tc1,
        out_shape=(
            jax.ShapeDtypeStruct((NP,), jnp.float32),
            jax.ShapeDtypeStruct((4, NP, 128), jnp.float32),
        ),
    )(deg2, x_p, W1p)
    hs1 = tab1[0, :, :F2]

    agg = _make_agg(NP, S)

    def decode(p):
        q = p.astype(jnp.float32)[:, :, _IPERM]
        return (q[0] + q[1]).reshape(NP, F2)

    p1 = decode(agg(tab1.reshape(4 * NP, 128), gidx, didx))
    y1, tab2 = pl.pallas_call(
        _tcstep,
        out_shape=(
            jax.ShapeDtypeStruct((NP, F2), jnp.float32),
            jax.ShapeDtypeStruct((4, NP, 128), jnp.float32),
        ),
    )(p1, hs1, dinv, b1p, W2p)
    hs2 = tab2[0, :, :F2]

    p2 = decode(agg(tab2.reshape(4 * NP, 128), gidx, didx))
    y2, _ = pl.pallas_call(
        _tcstep,
        out_shape=(
            jax.ShapeDtypeStruct((NP, F2), jnp.float32),
            jax.ShapeDtypeStruct((4, NP, 128), jnp.float32),
        ),
    )(p2, hs2, dinv, b2, W2p)

    out = pl.pallas_call(
        _tc3,
        out_shape=jax.ShapeDtypeStruct((G, 1), jnp.float32),
    )(y2, Wn, bn, batch_p, Wg, bg)

    return out


# speedup vs baseline: 1.0847x; 1.0847x over previous
import functools

import numpy as np

import jax
import jax.numpy as jnp
from jax import lax
from jax.experimental import pallas as pl
from jax.experimental.pallas import tpu as pltpu
from jax.experimental.pallas import tpu_sc as plsc

NC = 2
NS = 16
NW = NC * NS
EPB = 128
G = 16


def _sc_mesh():
    return plsc.VectorSubcoreMesh(
        core_axis_name="c", subcore_axis_name="s", num_cores=NC, num_subcores=NS
    )


def _make_degree(NP, S):
    @functools.partial(
        pl.kernel,
        out_type=jax.ShapeDtypeStruct((NW, NP), jnp.float32),
        mesh=_sc_mesh(),
        scratch_types=[
            pltpu.VMEM((S, EPB), jnp.int32),
            pltpu.VMEM((NP,), jnp.float32),
        ],
        compiler_params=pltpu.CompilerParams(needs_layout_passes=False),
    )
    def deg_kernel(dsts_hbm, out_hbm, idx_d, hist):
        c = lax.axis_index("c")
        s = lax.axis_index("s")
        wid = c * NS + s
        pltpu.sync_copy(dsts_hbm.at[wid], idx_d)

        def zbody(i, carry):
            hist[pl.ds(i * 16, 16)] = jnp.zeros((16,), jnp.float32)
            return carry

        lax.fori_loop(0, NP // 16, zbody, 0)
        ones16 = jnp.ones((16,), jnp.float32)

        def body(j, carry):
            for k in range(EPB // 16):
                idx = idx_d[j, pl.ds(k * 16, 16)]
                plsc.addupdate_scatter(hist, [idx], ones16)
            return carry

        lax.fori_loop(0, S, body, 0)
        pltpu.sync_copy(hist, out_hbm.at[wid])

    return deg_kernel


def _make_agg(NP, S):
    rq = NP // 4 // NS

    @functools.partial(
        pl.kernel,
        out_type=jax.ShapeDtypeStruct((NC, NP // 4, 128), jnp.bfloat16),
        mesh=_sc_mesh(),
        scratch_types=[
            pltpu.VMEM((S, EPB), jnp.int32),
            pltpu.VMEM((S, EPB), jnp.int32),
            pltpu.VMEM((EPB, 128), jnp.float32),
            pltpu.VMEM((EPB, 128), jnp.float32),
            pltpu.VMEM((NP // 4 // NS, 128), jnp.float32),
            pltpu.VMEM((NP // 4 // NS, 128), jnp.bfloat16),
            pltpu.VMEM_SHARED((NP // 4, 128), jnp.float32),
            pltpu.SemaphoreType.DMA,
            pltpu.SemaphoreType.DMA,
        ],
        compiler_params=pltpu.CompilerParams(needs_layout_passes=False),
    )
    def agg_kernel(tab_hbm, gidx_hbm, didx_hbm, out_hbm,
                   idx_g, idx_d, rows_a, rows_b, stage, stage_bf, acc,
                   sem_a, sem_b):
        c = lax.axis_index("c")
        s = lax.axis_index("s")
        wid = c * NS + s
        row0 = s * rq

        def zbody(i, carry):
            for k in range(8):
                stage[i, pl.ds(k * 16, 16)] = jnp.zeros((16,), jnp.float32)
            return carry

        lax.fori_loop(0, rq, zbody, 0)
        pltpu.sync_copy(stage, acc.at[pl.ds(row0, rq)])
        pltpu.sync_copy(gidx_hbm.at[wid], idx_g)
        pltpu.sync_copy(didx_hbm.at[wid], idx_d)
        plsc.subcore_barrier()

        pltpu.async_copy(tab_hbm.at[idx_g.at[0]], rows_a, sem_a)

        def body(t, carry):
            j0 = 2 * t
            j1 = 2 * t + 1
            pltpu.make_async_copy(tab_hbm.at[idx_g.at[j0]], rows_a, sem_a).wait()
            pltpu.async_copy(tab_hbm.at[idx_g.at[j1]], rows_b, sem_b)
            pltpu.sync_copy(rows_a, acc.at[idx_d.at[j0]], add=True)
            pltpu.make_async_copy(tab_hbm.at[idx_g.at[j1]], rows_b, sem_b).wait()

            @pl.when(t + 1 < S // 2)
            def _():
                pltpu.async_copy(tab_hbm.at[idx_g.at[j1 + 1]], rows_a, sem_a)

            pltpu.sync_copy(rows_b, acc.at[idx_d.at[j1]], add=True)
            return carry

        lax.fori_loop(0, S // 2, body, 0)
        plsc.subcore_barrier()
        pltpu.sync_copy(acc.at[pl.ds(row0, rq)], stage)

        def wbody(r, carry):
            for g in range(4):
                v0 = stage[r, pl.ds(g * 32, 16)]
                v1 = stage[r, pl.ds(g * 32 + 16, 16)]
                pk = plsc.pack(v0, v1, format=plsc.PackFormat.INTERLEAVED)
                stage_bf[r, pl.ds(g * 32, 32)] = pk
            return carry

        lax.fori_loop(0, rq, wbody, 0)
        pltpu.sync_copy(stage_bf, out_hbm.at[c, pl.ds(row0, rq)])

    return agg_kernel


def _tc1(deg2_ref, x_ref, w1_ref, dinv_ref, tab_ref):
    deg = jnp.sum(deg2_ref[...], axis=0) + 1.0
    dinv = lax.rsqrt(deg)
    dinv_ref[...] = dinv
    h = jnp.dot(x_ref[...], w1_ref[...], preferred_element_type=jnp.float32)
    hs = h * dinv[:, None]
    f = h.shape[1]
    for q in range(4):
        tab_ref[q] = jnp.pad(hs, ((0, 0), (q * f, 128 - (q + 1) * f)))


def _tcstep(p_ref, hs_ref, dinv_ref, b_ref, w_ref, y_ref, tabn_ref):
    dinv = dinv_ref[...]
    y = dinv[:, None] * (p_ref[...] + hs_ref[...]) + b_ref[...]
    y_ref[...] = y
    h = jnp.maximum(y, 0.0)
    hsn = jnp.dot(h, w_ref[...], preferred_element_type=jnp.float32) * dinv[:, None]
    f = hsn.shape[1]
    for q in range(4):
        tabn_ref[q] = jnp.pad(hsn, ((0, 0), (q * f, 128 - (q + 1) * f)))


def _tc3(y_ref, wn_ref, bn_ref, batch_ref, wg_ref, bg_ref, out_ref):
    node = jnp.dot(y_ref[...], wn_ref[...], preferred_element_type=jnp.float32) + bn_ref[0]
    npad = node.shape[0]
    seg = lax.broadcasted_iota(jnp.int32, (npad, G), 1)
    onehot = (batch_ref[...][:, None] == seg)
    ssum = jnp.sum(jnp.where(onehot, node, 0.0), axis=0)
    cnt = jnp.sum(jnp.where(onehot, 1.0, 0.0), axis=0)
    smax = jnp.max(jnp.where(onehot, node, -jnp.inf), axis=0)
    mean = ssum / jnp.maximum(cnt, 1.0)
    res = ssum * wg_ref[0, 0] + mean * wg_ref[1, 0] + smax * wg_ref[2, 0] + bg_ref[0]
    out_ref[...] = res[:, None]


_IPERM = np.concatenate(
    [32 * g + np.concatenate([2 * np.arange(16), 2 * np.arange(16) + 1])
     for g in range(4)]
)


def kernel(x, edge_index, batch, W1, b1, W2, b2, Wn, bn, Wg, bg):
    N, F_IN = x.shape
    E = edge_index.shape[1]
    F1 = W1.shape[1]
    F2 = W2.shape[1]

    NP = ((N // (NS * EPB)) + 1) * (NS * EPB)
    S = -(-E // (NW * EPB))
    S = ((S + 7) // 8) * 8
    EP = NW * S * EPB

    pad = EP - E
    pad_idx = (N + (jnp.arange(pad, dtype=jnp.int32) % (NP - N))).astype(jnp.int32)
    src_p = jnp.concatenate([edge_index[0], pad_idx])
    dst_p = jnp.concatenate([edge_index[1], pad_idx])
    gidx = ((dst_p % 4) * NP + src_p).reshape(NW, S, EPB)
    didx = (dst_p // 4).reshape(NW, S, EPB)
    dst_p = dst_p.reshape(NW, S, EPB)

    x_p = jnp.pad(x, ((0, NP - N), (0, 0)))
    batch_p = jnp.pad(batch, ((0, NP - N),), constant_values=G)
    W1p = jnp.pad(W1, ((0, 0), (0, F2 - F1)))
    b1p = jnp.pad(b1, ((0, F2 - F1)))
    W2p = jnp.pad(W2, ((0, F2 - F1), (0, 0)))

    deg2 = _make_degree(NP, S)(dst_p)

    dinv, tab1 = pl.pallas_call(
        _tc1,
        out_shape=(
            jax.ShapeDtypeStruct((NP,), jnp.float32),
            jax.ShapeDtypeStruct((4, NP, 128), jnp.float32),
        ),
    )(deg2, x_p, W1p)
    hs1 = tab1[0, :, :F2]

    agg = _make_agg(NP, S)

    def decode(p):
        q = p.astype(jnp.float32)[:, :, _IPERM]
        return (q[0] + q[1]).reshape(NP, F2)

    p1 = decode(agg(tab1.reshape(4 * NP, 128), gidx, didx))
    y1, tab2 = pl.pallas_call(
        _tcstep,
        out_shape=(
            jax.ShapeDtypeStruct((NP, F2), jnp.float32),
            jax.ShapeDtypeStruct((4, NP, 128), jnp.float32),
        ),
    )(p1, hs1, dinv, b1p, W2p)
    hs2 = tab2[0, :, :F2]

    p2 = decode(agg(tab2.reshape(4 * NP, 128), gidx, didx))
    y2, _ = pl.pallas_call(
        _tcstep,
        out_shape=(
            jax.ShapeDtypeStruct((NP, F2), jnp.float32),
            jax.ShapeDtypeStruct((4, NP, 128), jnp.float32),
        ),
    )(p2, hs2, dinv, b2, W2p)

    out = pl.pallas_call(
        _tc3,
        out_shape=jax.ShapeDtypeStruct((G, 1), jnp.float32),
    )(y2, Wn, bn, batch_p, Wg, bg)

    return out


# speedup vs baseline: 1.1303x; 1.0420x over previous
import functools

import numpy as np

import jax
import jax.numpy as jnp
from jax import lax
from jax.experimental import pallas as pl
from jax.experimental.pallas import tpu as pltpu
from jax.experimental.pallas import tpu_sc as plsc

NC = 2
NS = 16
NW = NC * NS
EPB = 128
G = 16


def _sc_mesh():
    return plsc.VectorSubcoreMesh(
        core_axis_name="c", subcore_axis_name="s", num_cores=NC, num_subcores=NS
    )


def _make_degree(NP, S):
    @functools.partial(
        pl.kernel,
        out_type=jax.ShapeDtypeStruct((NW, NP), jnp.float32),
        mesh=_sc_mesh(),
        scratch_types=[
            pltpu.VMEM((S, EPB), jnp.int32),
            pltpu.VMEM((NP,), jnp.float32),
        ],
        compiler_params=pltpu.CompilerParams(needs_layout_passes=False),
    )
    def deg_kernel(dsts_hbm, out_hbm, idx_d, hist):
        c = lax.axis_index("c")
        s = lax.axis_index("s")
        wid = c * NS + s
        pltpu.sync_copy(dsts_hbm.at[wid], idx_d)

        def zbody(i, carry):
            hist[pl.ds(i * 16, 16)] = jnp.zeros((16,), jnp.float32)
            return carry

        lax.fori_loop(0, NP // 16, zbody, 0)
        ones16 = jnp.ones((16,), jnp.float32)

        def body(j, carry):
            for k in range(EPB // 16):
                idx = idx_d[j, pl.ds(k * 16, 16)]
                plsc.addupdate_scatter(hist, [idx], ones16)
            return carry

        lax.fori_loop(0, S, body, 0)
        pltpu.sync_copy(hist, out_hbm.at[wid])

    return deg_kernel


def _make_agg(NP, S):
    rq = NP // 4 // NS

    @functools.partial(
        pl.kernel,
        out_type=jax.ShapeDtypeStruct((NC, NP // 4, 128), jnp.bfloat16),
        mesh=_sc_mesh(),
        scratch_types=[
            pltpu.VMEM((S, EPB), jnp.int32),
            pltpu.VMEM((S, EPB), jnp.int32),
            pltpu.VMEM((EPB, 128), jnp.float32),
            pltpu.VMEM((EPB, 128), jnp.float32),
            pltpu.VMEM((NP // 4 // NS, 128), jnp.float32),
            pltpu.VMEM((NP // 4 // NS, 128), jnp.bfloat16),
            pltpu.VMEM_SHARED((NP // 4, 128), jnp.float32),
            pltpu.SemaphoreType.DMA,
            pltpu.SemaphoreType.DMA,
        ],
        compiler_params=pltpu.CompilerParams(needs_layout_passes=False),
    )
    def agg_kernel(tab_hbm, gidx_hbm, didx_hbm, out_hbm,
                   idx_g, idx_d, rows_a, rows_b, stage, stage_bf, acc,
                   sem_a, sem_b):
        c = lax.axis_index("c")
        s = lax.axis_index("s")
        wid = c * NS + s
        row0 = s * rq

        def zbody(i, carry):
            for k in range(8):
                stage[i, pl.ds(k * 16, 16)] = jnp.zeros((16,), jnp.float32)
            return carry

        lax.fori_loop(0, rq, zbody, 0)
        pltpu.sync_copy(stage, acc.at[pl.ds(row0, rq)])
        pltpu.sync_copy(gidx_hbm.at[wid], idx_g)
        pltpu.sync_copy(didx_hbm.at[wid], idx_d)
        plsc.subcore_barrier()

        pltpu.async_copy(tab_hbm.at[idx_g.at[0]], rows_a, sem_a)

        def body(t, carry):
            j0 = 2 * t
            j1 = 2 * t + 1
            pltpu.make_async_copy(tab_hbm.at[idx_g.at[j0]], rows_a, sem_a).wait()
            pltpu.async_copy(tab_hbm.at[idx_g.at[j1]], rows_b, sem_b)
            pltpu.sync_copy(rows_a, acc.at[idx_d.at[j0]], add=True)
            pltpu.make_async_copy(tab_hbm.at[idx_g.at[j1]], rows_b, sem_b).wait()

            @pl.when(t + 1 < S // 2)
            def _():
                pltpu.async_copy(tab_hbm.at[idx_g.at[j1 + 1]], rows_a, sem_a)

            pltpu.sync_copy(rows_b, acc.at[idx_d.at[j1]], add=True)
            return carry

        lax.fori_loop(0, S // 2, body, 0)
        plsc.subcore_barrier()
        pltpu.sync_copy(acc.at[pl.ds(row0, rq)], stage)

        def wbody(r, carry):
            for g in range(4):
                v0 = stage[r, pl.ds(g * 32, 16)]
                v1 = stage[r, pl.ds(g * 32 + 16, 16)]
                pk = plsc.pack(v0, v1, format=plsc.PackFormat.INTERLEAVED)
                stage_bf[r, pl.ds(g * 32, 32)] = pk
            return carry

        lax.fori_loop(0, rq, wbody, 0)
        pltpu.sync_copy(stage_bf, out_hbm.at[c, pl.ds(row0, rq)])

    return agg_kernel


def _tc1(deg2_ref, x_ref, w1_ref, dinv_ref, tab_ref):
    deg = jnp.sum(deg2_ref[...], axis=0) + 1.0
    dinv = lax.rsqrt(deg)
    dinv_ref[...] = dinv
    h = jnp.dot(x_ref[...], w1_ref[...], preferred_element_type=jnp.float32)
    hs = h * dinv[:, None]
    f = h.shape[1]
    for q in range(4):
        tab_ref[q] = jnp.pad(hs, ((0, 0), (q * f, 128 - (q + 1) * f)))


def _tcstep(p_ref, hs_ref, dinv_ref, b_ref, w_ref, y_ref, tabn_ref):
    dinv = dinv_ref[...]
    y = dinv[:, None] * (p_ref[...] + hs_ref[...]) + b_ref[...]
    y_ref[...] = y
    h = jnp.maximum(y, 0.0)
    hsn = jnp.dot(h, w_ref[...], preferred_element_type=jnp.float32) * dinv[:, None]
    f = hsn.shape[1]
    for q in range(4):
        tabn_ref[q] = jnp.pad(hsn, ((0, 0), (q * f, 128 - (q + 1) * f)))


def _tc3(p_ref, hs_ref, dinv_ref, b_ref, wn_ref, bn_ref, batch_ref,
         wg_ref, bg_ref, out_ref):
    dinv = dinv_ref[...]
    y = dinv[:, None] * (p_ref[...] + hs_ref[...]) + b_ref[...]
    node = jnp.dot(y, wn_ref[...], preferred_element_type=jnp.float32) + bn_ref[0]
    npad = node.shape[0]
    seg = lax.broadcasted_iota(jnp.int32, (npad, G), 1)
    onehot = (batch_ref[...][:, None] == seg)
    ssum = jnp.sum(jnp.where(onehot, node, 0.0), axis=0)
    cnt = jnp.sum(jnp.where(onehot, 1.0, 0.0), axis=0)
    smax = jnp.max(jnp.where(onehot, node, -jnp.inf), axis=0)
    mean = ssum / jnp.maximum(cnt, 1.0)
    res = ssum * wg_ref[0, 0] + mean * wg_ref[1, 0] + smax * wg_ref[2, 0] + bg_ref[0]
    out_ref[...] = res[:, None]


_IPERM = np.concatenate(
    [32 * g + np.concatenate([2 * np.arange(16), 2 * np.arange(16) + 1])
     for g in range(4)]
)


def kernel(x, edge_index, batch, W1, b1, W2, b2, Wn, bn, Wg, bg):
    N, F_IN = x.shape
    E = edge_index.shape[1]
    F1 = W1.shape[1]
    F2 = W2.shape[1]

    NP = ((N // (NS * EPB)) + 1) * (NS * EPB)
    S = -(-E // (NW * EPB))
    S = ((S + 7) // 8) * 8
    EP = NW * S * EPB

    pad = EP - E
    pad_idx = (N + (jnp.arange(pad, dtype=jnp.int32) % (NP - N))).astype(jnp.int32)
    src_p = jnp.concatenate([edge_index[0], pad_idx])
    dst_p = jnp.concatenate([edge_index[1], pad_idx])
    gidx = ((dst_p % 4) * NP + src_p).reshape(NW, S, EPB)
    didx = (dst_p // 4).reshape(NW, S, EPB)
    dst_p = dst_p.reshape(NW, S, EPB)

    x_p = jnp.pad(x, ((0, NP - N), (0, 0)))
    batch_p = jnp.pad(batch, ((0, NP - N),), constant_values=G)
    W1p = jnp.pad(W1, ((0, 0), (0, F2 - F1)))
    b1p = jnp.pad(b1, ((0, F2 - F1)))
    W2p = jnp.pad(W2, ((0, F2 - F1), (0, 0)))

    deg2 = _make_degree(NP, S)(dst_p)

    dinv, tab1 = pl.pallas_call(
        _tc1,
        out_shape=(
            jax.ShapeDtypeStruct((NP,), jnp.float32),
            jax.ShapeDtypeStruct((4, NP, 128), jnp.float32),
        ),
    )(deg2, x_p, W1p)
    hs1 = tab1[0, :, :F2]

    agg = _make_agg(NP, S)

    def decode(p):
        q = p.astype(jnp.float32)[:, :, _IPERM]
        return (q[0] + q[1]).reshape(NP, F2)

    p1 = decode(agg(tab1.reshape(4 * NP, 128), gidx, didx))
    y1, tab2 = pl.pallas_call(
        _tcstep,
        out_shape=(
            jax.ShapeDtypeStruct((NP, F2), jnp.float32),
            jax.ShapeDtypeStruct((4, NP, 128), jnp.float32),
        ),
    )(p1, hs1, dinv, b1p, W2p)
    hs2 = tab2[0, :, :F2]

    p2 = decode(agg(tab2.reshape(4 * NP, 128), gidx, didx))
    out = pl.pallas_call(
        _tc3,
        out_shape=jax.ShapeDtypeStruct((G, 1), jnp.float32),
    )(p2, hs2, dinv, b2, Wn, bn, batch_p, Wg, bg)

    return out
